# Initial kernel scaffold; baseline (speedup 1.0000x reference)
import jax
import jax.numpy as jnp
from jax.experimental import pallas as pl


def kernel(x, steps_since_activation, W_enc, W_dec, input_bias, neuron_bias):
    raise NotImplementedError("write your pallas kernel here")


# speedup vs baseline: 1.0017x; 1.0017x over previous
import functools

import jax
import jax.numpy as jnp
from jax.experimental import pallas as pl
from jax.experimental.pallas import tpu as pltpu

B = 2048
D = 768
M = 32768
K = 32
AUX_K = 64
MULTI_K = 128
THRESH = 256

MBLK = 2048


def _encode_body(x_ref, w_ref, nb_ref, f_ref):
    acc = jax.lax.dot_general(
        x_ref[...], w_ref[...], (((1,), (1,)), ((), ())),
        preferred_element_type=jnp.float32,
    )
    f_ref[...] = jnp.maximum(acc + nb_ref[...], 0.0)


def _encode(x, W_enc, neuron_bias):
    grid = (M // MBLK,)
    return pl.pallas_call(
        _encode_body,
        grid=grid,
        in_specs=[
            pl.BlockSpec((B, D), lambda i: (0, 0)),
            pl.BlockSpec((MBLK, D), lambda i: (i, 0)),
            pl.BlockSpec((1, MBLK), lambda i: (0, i)),
        ],
        out_specs=pl.BlockSpec((B, MBLK), lambda i: (0, i)),
        out_shape=jax.ShapeDtypeStruct((B, M), jnp.float32),
    )(x, W_enc, neuron_bias.reshape(1, M))


def kernel(x, steps_since_activation, W_enc, W_dec, input_bias, neuron_bias):
    xc = x - input_bias
    f_full = _encode(xc, W_enc, neuron_bias)

    topk_values, topk_indices = jax.lax.top_k(f_full, K)
    rows = jnp.arange(B)[:, None]
    activations = jnp.zeros((B, M), jnp.float32).at[rows, topk_indices].set(topk_values)

    multik_values, multik_indices = jax.lax.top_k(f_full, MULTI_K)
    multik_activations = jnp.zeros((B, M), jnp.float32).at[rows, multik_indices].set(multik_values)

    steps = steps_since_activation + 1
    steps = steps.at[topk_indices.reshape(-1)].set(0)

    reconstruction = activations @ W_dec.T + input_bias
    multik_reconstruction = multik_activations @ W_dec.T + input_bias

    dead_mask = (steps > THRESH).astype(jnp.float32)
    dead_neuron_pre_act = (f_full + neuron_bias * 0.0) * dead_mask[None, :]
    aux_values, aux_indices = jax.lax.top_k(dead_neuron_pre_act, AUX_K)
    aux_values = jax.nn.relu(aux_values)

    return (reconstruction, activations, topk_values, topk_indices,
            multik_reconstruction, aux_values, aux_indices, f_full, steps)


# speedup vs baseline: 2.5074x; 2.5032x over previous
import functools

import jax
import jax.numpy as jnp
from jax import lax
from jax.experimental import pallas as pl
from jax.experimental.pallas import tpu as pltpu
from jax.experimental.pallas import tpu_sc as plsc

B = 2048
D = 768
M = 32768
K = 32
AUX_K = 64
MULTI_K = 128
THRESH = 256

MBLK = 2048
TILE = 128
NT = M // TILE
CAND = 512
CPAD = CAND + 16
NWORK = 32
FMAXBITS = 0x7F800000


def _encode_body(x_ref, w_ref, nb_ref, f_ref, tm_ref):
    acc = lax.dot_general(
        x_ref[...], w_ref[...], (((1,), (1,)), ((), ())),
        preferred_element_type=jnp.float32,
    )
    f = jnp.maximum(acc + nb_ref[...], 0.0)
    f_ref[...] = f
    parts = [jnp.max(f[:, t * TILE:(t + 1) * TILE], axis=1, keepdims=True)
             for t in range(MBLK // TILE)]
    tm_ref[0] = jnp.concatenate(parts, axis=1)


def _encode(x, W_enc, neuron_bias):
    return pl.pallas_call(
        _encode_body,
        grid=(M // MBLK,),
        in_specs=[
            pl.BlockSpec((B, D), lambda i: (0, 0)),
            pl.BlockSpec((MBLK, D), lambda i: (i, 0)),
            pl.BlockSpec((1, MBLK), lambda i: (0, i)),
        ],
        out_specs=[
            pl.BlockSpec((B, MBLK), lambda i: (0, i)),
            pl.BlockSpec((1, B, MBLK // TILE), lambda i: (i, 0, 0)),
        ],
        out_shape=[
            jax.ShapeDtypeStruct((B, M), jnp.float32),
            jax.ShapeDtypeStruct((M // MBLK, B, MBLK // TILE), jnp.float32),
        ],
    )(x, W_enc, neuron_bias.reshape(1, M))


def _thresh_body(tm_ref, taub_ref, *, rank):
    bits = lax.bitcast_convert_type(tm_ref[...], jnp.int32)
    lo0 = jnp.zeros((B, 1), jnp.int32)
    hi0 = jnp.full((B, 1), FMAXBITS, jnp.int32)

    def body(_, lohi):
        lo, hi = lohi
        mid = lo + lax.shift_right_logical(hi - lo + 1, 1)
        cnt = jnp.sum((bits >= mid).astype(jnp.int32), axis=1, keepdims=True)
        ok = cnt >= rank
        return jnp.where(ok, mid, lo), jnp.where(ok, hi, mid - 1)

    lo, _ = lax.fori_loop(0, 31, body, (lo0, hi0))
    tau = lax.bitcast_convert_type(lo, jnp.float32)
    taub_ref[...] = jnp.broadcast_to(tau, (B, 16))


def _thresholds(tm, rank):
    return pl.pallas_call(
        functools.partial(_thresh_body, rank=rank),
        out_shape=jax.ShapeDtypeStruct((B, 16), jnp.float32),
    )(tm)


def _compact_body(f_hbm, taub_hbm, mask_hbm, val_hbm, idx_hbm,
                  rowbuf, taubuf, maskbuf, valbuf, idxbuf, *, masked):
    cid = lax.axis_index("c")
    sid = lax.axis_index("s")
    wid = sid * 2 + cid
    rows_per = B // NWORK
    base_row = wid * rows_per

    if masked:
        pltpu.sync_copy(mask_hbm, maskbuf)

    iota = lax.iota(jnp.int32, 16)
    ones = jnp.ones((16,), jnp.int32)
    zeros = jnp.zeros((16,), jnp.int32)

    def row_body(r, _):
        row = base_row + r
        pltpu.sync_copy(f_hbm.at[row], rowbuf)
        pltpu.sync_copy(taub_hbm.at[row], taubuf)
        tau = taubuf[...]

        def init_body(i, c):
            valbuf[pl.ds(i * 16, 16)] = jnp.full((16,), -1.0, jnp.float32)
            idxbuf[pl.ds(i * 16, 16)] = jnp.zeros((16,), jnp.int32)
            return c
        lax.fori_loop(0, CPAD // 16, init_body, 0)

        def chunk_body(c, off):
            v = rowbuf[pl.ds(c * 16, 16)]
            if masked:
                v = v * maskbuf[pl.ds(c * 16, 16)]
            m = v >= tau
            cnt = jnp.sum(jnp.where(m, ones, zeros))

            @pl.when(cnt > 0)
            def _():
                plsc.store_compressed(valbuf.at[pl.ds(off, 16)], v, mask=m)
                plsc.store_compressed(idxbuf.at[pl.ds(off, 16)], iota + c * 16,
                                      mask=m)
            return jnp.minimum(off + cnt, CAND)

        lax.fori_loop(0, M // 16, chunk_body, jnp.int32(0))
        pltpu.sync_copy(valbuf, val_hbm.at[row])
        pltpu.sync_copy(idxbuf, idx_hbm.at[row])
        return _

    lax.fori_loop(0, B // NWORK, row_body, 0)


def _compact(f_full, taub, mask=None):
    masked = mask is not None
    if mask is None:
        mask = jnp.zeros((M,), jnp.float32)
    mesh = plsc.VectorSubcoreMesh(core_axis_name="c", subcore_axis_name="s")
    fn = pl.kernel(
        functools.partial(_compact_body, masked=masked),
        out_type=[
            jax.ShapeDtypeStruct((B, CPAD), jnp.float32),
            jax.ShapeDtypeStruct((B, CPAD), jnp.int32),
        ],
        mesh=mesh,
        compiler_params=pltpu.CompilerParams(needs_layout_passes=False),
        scratch_types=[
            pltpu.VMEM((M,), jnp.float32),
            pltpu.VMEM((16,), jnp.float32),
            pltpu.VMEM((M,), jnp.float32),
            pltpu.VMEM((CPAD,), jnp.float32),
            pltpu.VMEM((CPAD,), jnp.int32),
        ],
    )
    return fn(f_full, taub, mask)


def _crunch_body(cv_ref, ci_ref, tv_ref, ti_ref, bf_ref, bi_ref, *, k_out):
    vals0 = cv_ref[...]
    idxs = ci_ref[...]
    lane = lax.broadcasted_iota(jnp.int32, (1, k_out), 1)
    BIGI = jnp.int32(1 << 30)

    def ext_body(k, carry):
        vals, tv, ti = carry
        m = jnp.max(vals, axis=1, keepdims=True)
        sel = vals == m
        selidx = jnp.min(jnp.where(sel, idxs, BIGI), axis=1, keepdims=True)
        tv = jnp.where(lane == k, m, tv)
        ti = jnp.where(lane == k, selidx, ti)
        vals = jnp.where(sel & (idxs == selidx), -2.0, vals)
        return vals, tv, ti

    tv0 = jnp.zeros((B, k_out), jnp.float32)
    ti0 = jnp.zeros((B, k_out), jnp.int32)
    _, tv, ti = lax.fori_loop(0, k_out, ext_body, (vals0, tv0, ti0))
    tv_ref[...] = tv
    ti_ref[...] = ti

    bits = lax.bitcast_convert_type(vals0, jnp.int32)
    lo0 = jnp.zeros((B, 1), jnp.int32)
    hi0 = jnp.full((B, 1), FMAXBITS, jnp.int32)

    def vb(_, lohi):
        lo, hi = lohi
        mid = lo + lax.shift_right_logical(hi - lo + 1, 1)
        cnt = jnp.sum((bits >= mid).astype(jnp.int32), axis=1, keepdims=True)
        ok = cnt >= MULTI_K
        return jnp.where(ok, mid, lo), jnp.where(ok, hi, mid - 1)

    vlo, _ = lax.fori_loop(0, 31, vb, (lo0, hi0))
    v128 = lax.bitcast_convert_type(vlo, jnp.float32)
    cnt_gt = jnp.sum((bits >= vlo + 1).astype(jnp.int32), axis=1, keepdims=True)
    need = MULTI_K - cnt_gt
    eq = vals0 == v128

    lo2 = jnp.zeros((B, 1), jnp.int32)
    hi2 = jnp.full((B, 1), M - 1, jnp.int32)

    def ib(_, lohi):
        lo, hi = lohi
        mid = lax.shift_right_logical(lo + hi, 1)
        c = jnp.sum((eq & (idxs <= mid)).astype(jnp.int32), axis=1,
                    keepdims=True)
        ok = c >= need
        return jnp.where(ok, lo, mid + 1), jnp.where(ok, mid, hi)

    t128, _ = lax.fori_loop(0, 15, ib, (lo2, hi2))

    pad = jnp.zeros((B, 1), jnp.float32)
    padi = jnp.zeros((B, 1), jnp.int32)
    bf_ref[...] = jnp.concatenate(
        [v128, pad, pad, pad, pad, pad, pad, pad], axis=1)
    bi_ref[...] = jnp.concatenate(
        [t128, padi, padi, padi, padi, padi, padi, padi], axis=1)


def _crunch(cand_vals, cand_idx, k_out):
    return pl.pallas_call(
        functools.partial(_crunch_body, k_out=k_out),
        out_shape=[
            jax.ShapeDtypeStruct((B, k_out), jnp.float32),
            jax.ShapeDtypeStruct((B, k_out), jnp.int32),
            jax.ShapeDtypeStruct((B, 8), jnp.float32),
            jax.ShapeDtypeStruct((B, 8), jnp.int32),
        ],
    )(cand_vals, cand_idx)


def kernel(x, steps_since_activation, W_enc, W_dec, input_bias, neuron_bias):
    xc = x - input_bias
    f_full, tm3 = _encode(xc, W_enc, neuron_bias)
    tm = tm3.transpose(1, 0, 2).reshape(B, NT)
    taub = _thresholds(tm, MULTI_K)
    cand_vals, cand_idx = _compact(f_full, taub)
    topk_values, topk_indices, bf, bi = _crunch(cand_vals, cand_idx, K)
    v128 = bf[:, 0:1]
    t128 = bi[:, 0:1]
    v32 = topk_values[:, K - 1:K]
    t32 = topk_indices[:, K - 1:K]

    colidx = jnp.arange(M, dtype=jnp.int32)[None, :]
    member32 = (f_full > v32) | ((f_full == v32) & (colidx <= t32))
    member128 = (f_full > v128) | ((f_full == v128) & (colidx <= t128))
    activations = jnp.where(member32, f_full, 0.0)
    multik_activations = jnp.where(member128, f_full, 0.0)

    reset = jnp.any(member32, axis=0)
    steps = jnp.where(reset, 0, steps_since_activation + 1)

    reconstruction = activations @ W_dec.T + input_bias
    multik_reconstruction = multik_activations @ W_dec.T + input_bias

    dead_mask = (steps > THRESH).astype(jnp.float32)
    aux_values, aux_indices = jax.lax.top_k(f_full * dead_mask[None, :], AUX_K)

    return (reconstruction, activations, topk_values, topk_indices,
            multik_reconstruction, aux_values, aux_indices, f_full, steps)


# speedup vs baseline: 6.2481x; 2.4919x over previous
import functools

import jax
import jax.numpy as jnp
from jax import lax
from jax.experimental import pallas as pl
from jax.experimental.pallas import tpu as pltpu
from jax.experimental.pallas import tpu_sc as plsc

B = 2048
D = 768
M = 32768
K = 32
AUX_K = 64
MULTI_K = 128
THRESH = 256

MBLK = 2048
TILE = 128
NT = M // TILE
CAND = 512
CPAD = CAND + 16
NWORK = 32
FMAXBITS = 0x7F800000


def _encode_body(x_ref, w_ref, nb_ref, f_ref, tm_ref):
    acc = lax.dot_general(
        x_ref[...], w_ref[...], (((1,), (1,)), ((), ())),
        preferred_element_type=jnp.float32,
    )
    f = jnp.maximum(acc + nb_ref[...], 0.0)
    f_ref[...] = f
    parts = [jnp.max(f[:, t * TILE:(t + 1) * TILE], axis=1, keepdims=True)
             for t in range(MBLK // TILE)]
    tm_ref[0] = jnp.concatenate(parts, axis=1)


def _encode(x, W_enc, neuron_bias):
    return pl.pallas_call(
        _encode_body,
        grid=(M // MBLK,),
        in_specs=[
            pl.BlockSpec((B, D), lambda i: (0, 0)),
            pl.BlockSpec((MBLK, D), lambda i: (i, 0)),
            pl.BlockSpec((1, MBLK), lambda i: (0, i)),
        ],
        out_specs=[
            pl.BlockSpec((B, MBLK), lambda i: (0, i)),
            pl.BlockSpec((1, B, MBLK // TILE), lambda i: (i, 0, 0)),
        ],
        out_shape=[
            jax.ShapeDtypeStruct((B, M), jnp.float32),
            jax.ShapeDtypeStruct((M // MBLK, B, MBLK // TILE), jnp.float32),
        ],
    )(x, W_enc, neuron_bias.reshape(1, M))


def _thresh_body(tm_ref, taub_ref, *, rank):
    bits = lax.bitcast_convert_type(tm_ref[...], jnp.int32)
    lo0 = jnp.zeros((B, 1), jnp.int32)
    hi0 = jnp.full((B, 1), FMAXBITS, jnp.int32)

    def body(_, lohi):
        lo, hi = lohi
        mid = lo + lax.shift_right_logical(hi - lo + 1, 1)
        cnt = jnp.sum((bits >= mid).astype(jnp.int32), axis=1, keepdims=True)
        ok = cnt >= rank
        return jnp.where(ok, mid, lo), jnp.where(ok, hi, mid - 1)

    lo, _ = lax.fori_loop(0, 31, body, (lo0, hi0))
    tau = lax.bitcast_convert_type(lo, jnp.float32)
    taub_ref[...] = jnp.broadcast_to(tau, (B, 16))


def _thresholds(tm, rank):
    return pl.pallas_call(
        functools.partial(_thresh_body, rank=rank),
        out_shape=jax.ShapeDtypeStruct((B, 16), jnp.float32),
    )(tm)


def _compact_body(f_hbm, taub_hbm, mask_hbm, val_hbm, idx_hbm,
                  rowbuf, taubuf, maskbuf, valbuf, idxbuf, *, masked):
    cid = lax.axis_index("c")
    sid = lax.axis_index("s")
    wid = sid * 2 + cid
    rows_per = B // NWORK
    base_row = wid * rows_per

    if masked:
        pltpu.sync_copy(mask_hbm, maskbuf)

    iota = lax.iota(jnp.int32, 16)
    ones = jnp.ones((16,), jnp.int32)
    zeros = jnp.zeros((16,), jnp.int32)

    def row_body(r, _):
        row = base_row + r
        pltpu.sync_copy(f_hbm.at[row], rowbuf)
        pltpu.sync_copy(taub_hbm.at[row], taubuf)
        tau = taubuf[...]

        def init_body(i, c):
            valbuf[pl.ds(i * 16, 16)] = jnp.full((16,), -1.0, jnp.float32)
            idxbuf[pl.ds(i * 16, 16)] = jnp.zeros((16,), jnp.int32)
            return c
        lax.fori_loop(0, CPAD // 16, init_body, 0)

        def chunk_body(c, off):
            v = rowbuf[pl.ds(c * 16, 16)]
            if masked:
                v = v * maskbuf[pl.ds(c * 16, 16)]
            m = v >= tau
            cnt = jnp.sum(jnp.where(m, ones, zeros))

            @pl.when(cnt > 0)
            def _():
                plsc.store_compressed(valbuf.at[pl.ds(off, 16)], v, mask=m)
                plsc.store_compressed(idxbuf.at[pl.ds(off, 16)], iota + c * 16,
                                      mask=m)
            return jnp.minimum(off + cnt, CAND)

        lax.fori_loop(0, M // 16, chunk_body, jnp.int32(0))
        pltpu.sync_copy(valbuf, val_hbm.at[row])
        pltpu.sync_copy(idxbuf, idx_hbm.at[row])
        return _

    lax.fori_loop(0, B // NWORK, row_body, 0)


def _compact(f_full, taub, mask=None):
    masked = mask is not None
    if mask is None:
        mask = jnp.zeros((M,), jnp.float32)
    mesh = plsc.VectorSubcoreMesh(core_axis_name="c", subcore_axis_name="s")
    fn = pl.kernel(
        functools.partial(_compact_body, masked=masked),
        out_type=[
            jax.ShapeDtypeStruct((B, CPAD), jnp.float32),
            jax.ShapeDtypeStruct((B, CPAD), jnp.int32),
        ],
        mesh=mesh,
        compiler_params=pltpu.CompilerParams(needs_layout_passes=False),
        scratch_types=[
            pltpu.VMEM((M,), jnp.float32),
            pltpu.VMEM((16,), jnp.float32),
            pltpu.VMEM((M,), jnp.float32),
            pltpu.VMEM((CPAD,), jnp.float32),
            pltpu.VMEM((CPAD,), jnp.int32),
        ],
    )
    return fn(f_full, taub, mask)


def _crunch_body(cv_ref, ci_ref, tv_ref, ti_ref, bf_ref, bi_ref, *, k_out):
    vals0 = cv_ref[...]
    idxs = ci_ref[...]
    lane = lax.broadcasted_iota(jnp.int32, (1, k_out), 1)
    BIGI = jnp.int32(1 << 30)

    def ext_body(k, carry):
        vals, tv, ti = carry
        m = jnp.max(vals, axis=1, keepdims=True)
        sel = vals == m
        selidx = jnp.min(jnp.where(sel, idxs, BIGI), axis=1, keepdims=True)
        tv = jnp.where(lane == k, m, tv)
        ti = jnp.where(lane == k, selidx, ti)
        vals = jnp.where(sel & (idxs == selidx), -2.0, vals)
        return vals, tv, ti

    tv0 = jnp.zeros((B, k_out), jnp.float32)
    ti0 = jnp.zeros((B, k_out), jnp.int32)
    _, tv, ti = lax.fori_loop(0, k_out, ext_body, (vals0, tv0, ti0))
    tv_ref[...] = tv
    ti_ref[...] = ti

    bits = lax.bitcast_convert_type(vals0, jnp.int32)
    lo0 = jnp.zeros((B, 1), jnp.int32)
    hi0 = jnp.full((B, 1), FMAXBITS, jnp.int32)

    def vb(_, lohi):
        lo, hi = lohi
        mid = lo + lax.shift_right_logical(hi - lo + 1, 1)
        cnt = jnp.sum((bits >= mid).astype(jnp.int32), axis=1, keepdims=True)
        ok = cnt >= MULTI_K
        return jnp.where(ok, mid, lo), jnp.where(ok, hi, mid - 1)

    vlo, _ = lax.fori_loop(0, 31, vb, (lo0, hi0))
    v128 = lax.bitcast_convert_type(vlo, jnp.float32)
    cnt_gt = jnp.sum((bits >= vlo + 1).astype(jnp.int32), axis=1, keepdims=True)
    need = MULTI_K - cnt_gt
    eq = vals0 == v128

    lo2 = jnp.zeros((B, 1), jnp.int32)
    hi2 = jnp.full((B, 1), M - 1, jnp.int32)

    def ib(_, lohi):
        lo, hi = lohi
        mid = lax.shift_right_logical(lo + hi, 1)
        c = jnp.sum((eq & (idxs <= mid)).astype(jnp.int32), axis=1,
                    keepdims=True)
        ok = c >= need
        return jnp.where(ok, lo, mid + 1), jnp.where(ok, mid, hi)

    t128, _ = lax.fori_loop(0, 15, ib, (lo2, hi2))

    pad = jnp.zeros((B, 1), jnp.float32)
    padi = jnp.zeros((B, 1), jnp.int32)
    bf_ref[...] = jnp.concatenate(
        [v128, pad, pad, pad, pad, pad, pad, pad], axis=1)
    bi_ref[...] = jnp.concatenate(
        [t128, padi, padi, padi, padi, padi, padi, padi], axis=1)


def _crunch(cand_vals, cand_idx, k_out):
    return pl.pallas_call(
        functools.partial(_crunch_body, k_out=k_out),
        out_shape=[
            jax.ShapeDtypeStruct((B, k_out), jnp.float32),
            jax.ShapeDtypeStruct((B, k_out), jnp.int32),
            jax.ShapeDtypeStruct((B, 8), jnp.float32),
            jax.ShapeDtypeStruct((B, 8), jnp.int32),
        ],
    )(cand_vals, cand_idx)


DBLK = 512


def _decode_body(bf_ref, bi_ref, ib_ref, f_ref, w_ref, st_ref,
                 act_ref, steps_ref, rec_ref, mrec_ref, acc1, acc2):
    i = pl.program_id(0)
    nsteps = pl.num_programs(0)
    f = f_ref[...]
    v32 = bf_ref[:, 0:1]
    v128 = bf_ref[:, 1:2]
    t32 = bi_ref[:, 0:1]
    t128 = bi_ref[:, 1:2]
    col = lax.broadcasted_iota(jnp.int32, (1, DBLK), 1) + i * DBLK
    m32 = (f > v32) | ((f == v32) & (col <= t32))
    m128 = (f > v128) | ((f == v128) & (col <= t128))
    act = jnp.where(m32, f, 0.0)
    mact = jnp.where(m128, f, 0.0)
    act_ref[...] = act
    reset = jnp.max(jnp.where(m32, 1, 0), axis=0, keepdims=True)
    steps_ref[...] = jnp.where(reset > 0, 0, st_ref[...] + 1)
    w = w_ref[...]
    pa = lax.dot_general(act, w, (((1,), (1,)), ((), ())),
                         preferred_element_type=jnp.float32)
    pb = lax.dot_general(mact, w, (((1,), (1,)), ((), ())),
                         preferred_element_type=jnp.float32)

    @pl.when(i == 0)
    def _():
        acc1[...] = pa
        acc2[...] = pb

    @pl.when(i > 0)
    def _():
        acc1[...] += pa
        acc2[...] += pb

    @pl.when(i == nsteps - 1)
    def _():
        rec_ref[...] = acc1[...] + ib_ref[...]
        mrec_ref[...] = acc2[...] + ib_ref[...]


def _decode(bf, bi, input_bias, f_full, W_dec, steps_in):
    return pl.pallas_call(
        _decode_body,
        grid=(M // DBLK,),
        in_specs=[
            pl.BlockSpec((B, 8), lambda i: (0, 0)),
            pl.BlockSpec((B, 8), lambda i: (0, 0)),
            pl.BlockSpec((1, D), lambda i: (0, 0)),
            pl.BlockSpec((B, DBLK), lambda i: (0, i)),
            pl.BlockSpec((D, DBLK), lambda i: (0, i)),
            pl.BlockSpec((1, DBLK), lambda i: (0, i)),
        ],
        out_specs=[
            pl.BlockSpec((B, DBLK), lambda i: (0, i)),
            pl.BlockSpec((1, DBLK), lambda i: (0, i)),
            pl.BlockSpec((B, D), lambda i: (0, 0)),
            pl.BlockSpec((B, D), lambda i: (0, 0)),
        ],
        out_shape=[
            jax.ShapeDtypeStruct((B, M), jnp.float32),
            jax.ShapeDtypeStruct((1, M), jnp.int32),
            jax.ShapeDtypeStruct((B, D), jnp.float32),
            jax.ShapeDtypeStruct((B, D), jnp.float32),
        ],
        scratch_shapes=[
            pltpu.VMEM((B, D), jnp.float32),
            pltpu.VMEM((B, D), jnp.float32),
        ],
    )(bf, bi, input_bias.reshape(1, D), f_full, W_dec,
      steps_in.reshape(1, M))


def _auxtm_body(f_ref, st_ref, tm_ref, dead_ref):
    dead = (st_ref[...] > THRESH).astype(jnp.float32)
    dead_ref[...] = dead
    mf = f_ref[...] * dead
    parts = [jnp.max(mf[:, t * TILE:(t + 1) * TILE], axis=1, keepdims=True)
             for t in range(MBLK // TILE)]
    tm_ref[0] = jnp.concatenate(parts, axis=1)


def _auxtm(f_full, steps):
    return pl.pallas_call(
        _auxtm_body,
        grid=(M // MBLK,),
        in_specs=[
            pl.BlockSpec((B, MBLK), lambda i: (0, i)),
            pl.BlockSpec((1, MBLK), lambda i: (0, i)),
        ],
        out_specs=[
            pl.BlockSpec((1, B, MBLK // TILE), lambda i: (i, 0, 0)),
            pl.BlockSpec((1, MBLK), lambda i: (0, i)),
        ],
        out_shape=[
            jax.ShapeDtypeStruct((M // MBLK, B, MBLK // TILE), jnp.float32),
            jax.ShapeDtypeStruct((1, M), jnp.float32),
        ],
    )(f_full, steps.reshape(1, M))


def kernel(x, steps_since_activation, W_enc, W_dec, input_bias, neuron_bias):
    xc = x - input_bias
    f_full, tm3 = _encode(xc, W_enc, neuron_bias)
    tm = tm3.transpose(1, 0, 2).reshape(B, NT)
    taub = _thresholds(tm, MULTI_K)
    cand_vals, cand_idx = _compact(f_full, taub)
    topk_values, topk_indices, bf, bi = _crunch(cand_vals, cand_idx, K)
    v32 = topk_values[:, K - 1:K]
    t32 = topk_indices[:, K - 1:K]
    zf = jnp.zeros((B, 6), jnp.float32)
    zi = jnp.zeros((B, 6), jnp.int32)
    bf5 = jnp.concatenate([v32, bf[:, 0:1], zf], axis=1)
    bi5 = jnp.concatenate([t32, bi[:, 0:1], zi], axis=1)

    activations, steps2d, reconstruction, multik_reconstruction = _decode(
        bf5, bi5, input_bias, f_full, W_dec, steps_since_activation)
    steps = steps2d.reshape(M)

    tm_aux3, dead_f = _auxtm(f_full, steps)
    tm_aux = tm_aux3.transpose(1, 0, 2).reshape(B, NT)
    taub_aux = _thresholds(tm_aux, AUX_K)
    aux_cv, aux_ci = _compact(f_full, taub_aux, mask=dead_f.reshape(M))
    aux_values, aux_indices, _, _ = _crunch(aux_cv, aux_ci, AUX_K)

    return (reconstruction, activations, topk_values, topk_indices,
            multik_reconstruction, aux_values, aux_indices, f_full, steps)
